# baseline (speedup 1.0000x reference)
import functools

import jax
import jax.numpy as jnp
from jax import lax
from jax.experimental import pallas as pl
from jax.experimental.pallas import tpu as pltpu
from jax.experimental.pallas import tpu_sc as plsc

_NC, _NS, _LANES = 2, 16, 16
_NW = _NC * _NS

_LVL_BITS = (8, 8, 8, 7)
_LVL_SHIFTS = (23, 15, 7, 0)
_NCHUNK = 8


def _bucket16(v, shift, nbits):
    m16 = ((1 << nbits) - 1) << 4
    if shift >= 4:
        return (v >> (shift - 4)) & jnp.int32(m16)
    return (v << (4 - shift)) & jnp.int32(m16)


def _sc_select_body(k, nvec, x_hbm, thr_hbm, xbuf0, xbuf1, hist0, hist1,
                    tvec, *sems):
    lane = lax.iota(jnp.int32, _LANES)
    ones = jnp.full((_LANES,), 1, jnp.int32)
    zeros16 = jnp.zeros((_LANES,), jnp.int32)
    wid = lax.axis_index("c") * _NS + lax.axis_index("s")

    nelem = nvec * _LANES
    csize = nelem // _NCHUNK
    copies = []
    for c in range(_NCHUNK):
        copies.append(
            (pltpu.async_copy(x_hbm.at[wid * 2, pl.ds(c * csize, csize)],
                              xbuf0.at[pl.ds(c * csize, csize)], sems[c]),
             pltpu.async_copy(x_hbm.at[wid * 2 + 1, pl.ds(c * csize, csize)],
                              xbuf1.at[pl.ds(c * csize, csize)], sems[c])))

    kk0 = jnp.int32(k)
    kk1 = jnp.int32(k)
    p0 = jnp.int32(0)
    p1 = jnp.int32(0)
    for lvl in range(4):
        shift = _LVL_SHIFTS[lvl]
        nbits = _LVL_BITS[lvl]
        nb = 1 << nbits

        @plsc.parallel_loop(0, nb, unroll=8)
        def _(b):
            hist0[pl.ds(b * _LANES, _LANES)] = zeros16
            hist1[pl.ds(b * _LANES, _LANES)] = zeros16

        pfx0, pfx1 = p0, p1
        mlow = jnp.int32((1 << (31 - shift - nbits)) - 1)

        def span(i0v, i1v, lvl=lvl, shift=shift, nbits=nbits, nb=nb,
                 pfx0=pfx0, pfx1=pfx1, mlow=mlow):
            @plsc.parallel_loop(i0v, i1v, unroll=16)
            def _(i):
                v0 = plsc.bitcast(xbuf0[pl.ds(i * _LANES, _LANES)], jnp.int32)
                v1 = plsc.bitcast(xbuf1[pl.ds(i * _LANES, _LANES)], jnp.int32)
                i0 = _bucket16(v0, shift, nbits) + lane
                i1 = _bucket16(v1, shift, nbits) + lane
                if lvl == 0:
                    plsc.addupdate_scatter(hist0, [i0], ones)
                    plsc.addupdate_scatter(hist1, [i1], ones)
                else:
                    s0 = ((v0 >> (shift + nbits)) & mlow) == pfx0
                    s1 = ((v1 >> (shift + nbits)) & mlow) == pfx1
                    plsc.addupdate_scatter(hist0, [i0], ones, mask=s0)
                    plsc.addupdate_scatter(hist1, [i1], ones, mask=s1)

        if lvl == 0:
            cvec = csize // _LANES
            for c in range(_NCHUNK):
                copies[c][0].wait()
                copies[c][1].wait()
                span(c * cvec, (c + 1) * cvec)
        else:
            span(0, nvec)

        @plsc.parallel_loop(0, nb, carry=(zeros16, zeros16))
        def _(t, acc, nb=nb):
            a0, a1 = acc
            bb = nb - 1 - t
            a0 = a0 + hist0[pl.ds(bb * _LANES, _LANES)]
            a1 = a1 + hist1[pl.ds(bb * _LANES, _LANES)]
            hist0[pl.ds(bb * _LANES, _LANES)] = a0
            hist1[pl.ds(bb * _LANES, _LANES)] = a1
            return (a0, a1)

        def probe(hist, b):
            return jnp.sum(hist[pl.ds(b * _LANES, _LANES)])

        lo0 = jnp.int32(0)
        lo1 = jnp.int32(0)
        step = nb >> 1
        while step >= 1:
            c0 = lo0 + jnp.int32(step)
            c1 = lo1 + jnp.int32(step)
            sv0 = probe(hist0, jnp.minimum(c0, nb - 1))
            sv1 = probe(hist1, jnp.minimum(c1, nb - 1))
            lo0 = jnp.where((c0 <= nb - 1) & (sv0 >= kk0), c0, lo0)
            lo1 = jnp.where((c1 <= nb - 1) & (sv1 >= kk1), c1, lo1)
            step >>= 1
        a0 = probe(hist0, jnp.minimum(lo0 + 1, nb - 1))
        a1 = probe(hist1, jnp.minimum(lo1 + 1, nb - 1))
        kk0 = kk0 - jnp.where(lo0 + 1 <= nb - 1, a0, jnp.int32(0))
        kk1 = kk1 - jnp.where(lo1 + 1 <= nb - 1, a1, jnp.int32(0))
        p0 = (p0 << nbits) | lo0
        p1 = (p1 << nbits) | lo1
    tvec[...] = jnp.broadcast_to(p0, (_LANES,))
    pltpu.sync_copy(tvec, thr_hbm.at[wid * 2])
    tvec[...] = jnp.broadcast_to(p1, (_LANES,))
    pltpu.sync_copy(tvec, thr_hbm.at[wid * 2 + 1])


def _sc_select(x, k):
    B, H = x.shape
    mesh = plsc.VectorSubcoreMesh(
        core_axis_name="c", subcore_axis_name="s", num_cores=_NC,
        num_subcores=_NS)
    body = functools.partial(_sc_select_body, k, H // _LANES)
    nbmax = 1 << max(_LVL_BITS)
    return pl.kernel(
        body,
        out_type=jax.ShapeDtypeStruct((B, _LANES), jnp.int32),
        mesh=mesh,
        compiler_params=pltpu.CompilerParams(needs_layout_passes=False),
        scratch_types=[
            pltpu.VMEM((H,), jnp.float32),
            pltpu.VMEM((H,), jnp.float32),
            pltpu.VMEM((nbmax * _LANES,), jnp.int32),
            pltpu.VMEM((nbmax * _LANES,), jnp.int32),
            pltpu.VMEM((_LANES,), jnp.int32),
        ] + [pltpu.SemaphoreType.DMA] * _NCHUNK,
    )(x)


def _apply_body(thr_ref, x_ref, sparse_ref, mask_ref):
    x = x_ref[...]
    u = lax.bitcast_convert_type(jnp.abs(x), jnp.int32)
    keep = u >= thr_ref[:, 0:1]
    mask_ref[...] = keep.astype(jnp.int8)
    sparse_ref[...] = jnp.where(keep, x, 0.0)


def _tc_apply(x, thr, rows):
    B, H = x.shape
    return pl.pallas_call(
        _apply_body,
        grid=(B // rows,),
        in_specs=[
            pl.BlockSpec((rows, _LANES), lambda i: (i, 0)),
            pl.BlockSpec((rows, H), lambda i: (i, 0)),
        ],
        out_specs=[
            pl.BlockSpec((rows, H), lambda i: (i, 0)),
            pl.BlockSpec((rows, H), lambda i: (i, 0)),
        ],
        out_shape=[
            jax.ShapeDtypeStruct((B, H), jnp.float32),
            jax.ShapeDtypeStruct((B, H), jnp.int8),
        ],
    )(thr, x)


def _select_body(x_ref, sparse_ref, mask_ref, *, k):
    x = x_ref[...]
    u = lax.bitcast_convert_type(jnp.abs(x), jnp.int32)

    def step(i, p):
        cand = p | (jnp.int32(1) << (30 - i))
        cnt = jnp.sum((u >= cand).astype(jnp.int32), axis=1, keepdims=True)
        return jnp.where(cnt >= k, cand, p)

    p0 = jnp.zeros((x.shape[0], 1), jnp.int32)
    thr = lax.fori_loop(0, 31, step, p0)
    keep = u >= thr
    mask_ref[...] = keep
    sparse_ref[...] = jnp.where(keep, x, 0.0)


def _tc_only(flat, k):
    B, H = flat.shape
    rows = 8 if B % 8 == 0 else 1
    return pl.pallas_call(
        functools.partial(_select_body, k=k),
        grid=(B // rows,),
        in_specs=[pl.BlockSpec((rows, H), lambda i: (i, 0))],
        out_specs=[
            pl.BlockSpec((rows, H), lambda i: (i, 0)),
            pl.BlockSpec((rows, H), lambda i: (i, 0)),
        ],
        out_shape=[
            jax.ShapeDtypeStruct((B, H), jnp.float32),
            jax.ShapeDtypeStruct((B, H), jnp.bool_),
        ],
    )(flat)


def kernel(x):
    flat = x if x.ndim == 2 else x.reshape(x.shape[0], -1)
    B, H = flat.shape
    k = max(1, int(H * 10.0 / 100.0))
    if B == 2 * _NW and H % (_LANES * _NCHUNK) == 0 and B % 32 == 0:
        thr = _sc_select(flat, k)
        sparse, mask8 = _tc_apply(flat, thr, 32)
        mask = mask8.astype(jnp.bool_)
    else:
        sparse, mask = _tc_only(flat, k)
    return sparse.reshape(x.shape), mask.reshape(x.shape)


# speedup vs baseline: 1.1991x; 1.1991x over previous
import functools

import jax
import jax.numpy as jnp
from jax import lax
from jax.experimental import pallas as pl
from jax.experimental.pallas import tpu as pltpu
from jax.experimental.pallas import tpu_sc as plsc

_NC, _NS, _LANES = 2, 16, 16
_NW = _NC * _NS

_LVL_BITS = (8, 8, 8, 7)
_LVL_SHIFTS = (23, 15, 7, 0)
_NCHUNK = 4


def _bucket16(v, shift, nbits):
    m16 = ((1 << nbits) - 1) << 4
    if shift >= 4:
        return (v >> (shift - 4)) & jnp.int32(m16)
    return (v << (4 - shift)) & jnp.int32(m16)


def _sc_select_body(k, nvec, x_hbm, thr_hbm, xbuf0, xbuf1, hist0, hist1,
                    tvec, *sems):
    lane = lax.iota(jnp.int32, _LANES)
    ones = jnp.full((_LANES,), 1, jnp.int32)
    zeros16 = jnp.zeros((_LANES,), jnp.int32)
    wid = lax.axis_index("c") * _NS + lax.axis_index("s")

    nelem = nvec * _LANES
    csize = nelem // _NCHUNK
    copies = []
    for c in range(_NCHUNK):
        copies.append(
            (pltpu.async_copy(x_hbm.at[wid * 2, pl.ds(c * csize, csize)],
                              xbuf0.at[pl.ds(c * csize, csize)], sems[c]),
             pltpu.async_copy(x_hbm.at[wid * 2 + 1, pl.ds(c * csize, csize)],
                              xbuf1.at[pl.ds(c * csize, csize)], sems[c])))

    kk0 = jnp.int32(k)
    kk1 = jnp.int32(k)
    p0 = jnp.int32(0)
    p1 = jnp.int32(0)
    for lvl in range(4):
        shift = _LVL_SHIFTS[lvl]
        nbits = _LVL_BITS[lvl]
        nb = 1 << nbits

        @plsc.parallel_loop(0, nb, unroll=8)
        def _(b):
            hist0[pl.ds(b * _LANES, _LANES)] = zeros16
            hist1[pl.ds(b * _LANES, _LANES)] = zeros16

        pfx0, pfx1 = p0, p1
        mlow = jnp.int32((1 << (31 - shift - nbits)) - 1)

        def span(i0v, i1v, lvl=lvl, shift=shift, nbits=nbits, nb=nb,
                 pfx0=pfx0, pfx1=pfx1, mlow=mlow):
            @plsc.parallel_loop(i0v, i1v, unroll=8)
            def _(i):
                v0 = plsc.bitcast(xbuf0[pl.ds(i * _LANES, _LANES)], jnp.int32)
                v1 = plsc.bitcast(xbuf1[pl.ds(i * _LANES, _LANES)], jnp.int32)
                i0 = _bucket16(v0, shift, nbits) + lane
                i1 = _bucket16(v1, shift, nbits) + lane
                if lvl == 0:
                    plsc.addupdate_scatter(hist0, [i0], ones)
                    plsc.addupdate_scatter(hist1, [i1], ones)
                else:
                    s0 = ((v0 >> (shift + nbits)) & mlow) == pfx0
                    s1 = ((v1 >> (shift + nbits)) & mlow) == pfx1
                    plsc.addupdate_scatter(hist0, [i0], ones, mask=s0)
                    plsc.addupdate_scatter(hist1, [i1], ones, mask=s1)

        if lvl == 0:
            cvec = csize // _LANES
            for c in range(_NCHUNK):
                copies[c][0].wait()
                copies[c][1].wait()
                span(c * cvec, (c + 1) * cvec)
        else:
            span(0, nvec)

        @plsc.parallel_loop(0, nb, carry=(zeros16, zeros16))
        def _(t, acc, nb=nb):
            a0, a1 = acc
            bb = nb - 1 - t
            a0 = a0 + hist0[pl.ds(bb * _LANES, _LANES)]
            a1 = a1 + hist1[pl.ds(bb * _LANES, _LANES)]
            hist0[pl.ds(bb * _LANES, _LANES)] = a0
            hist1[pl.ds(bb * _LANES, _LANES)] = a1
            return (a0, a1)

        def probe(hist, b):
            return jnp.sum(hist[pl.ds(b * _LANES, _LANES)])

        lo0 = jnp.int32(0)
        lo1 = jnp.int32(0)
        step = nb >> 1
        while step >= 1:
            c0 = lo0 + jnp.int32(step)
            c1 = lo1 + jnp.int32(step)
            sv0 = probe(hist0, jnp.minimum(c0, nb - 1))
            sv1 = probe(hist1, jnp.minimum(c1, nb - 1))
            lo0 = jnp.where((c0 <= nb - 1) & (sv0 >= kk0), c0, lo0)
            lo1 = jnp.where((c1 <= nb - 1) & (sv1 >= kk1), c1, lo1)
            step >>= 1
        a0 = probe(hist0, jnp.minimum(lo0 + 1, nb - 1))
        a1 = probe(hist1, jnp.minimum(lo1 + 1, nb - 1))
        kk0 = kk0 - jnp.where(lo0 + 1 <= nb - 1, a0, jnp.int32(0))
        kk1 = kk1 - jnp.where(lo1 + 1 <= nb - 1, a1, jnp.int32(0))
        p0 = (p0 << nbits) | lo0
        p1 = (p1 << nbits) | lo1
    tvec[...] = jnp.broadcast_to(p0, (_LANES,))
    pltpu.sync_copy(tvec, thr_hbm.at[wid * 2])
    tvec[...] = jnp.broadcast_to(p1, (_LANES,))
    pltpu.sync_copy(tvec, thr_hbm.at[wid * 2 + 1])


def _sc_select(x, k):
    B, H = x.shape
    mesh = plsc.VectorSubcoreMesh(
        core_axis_name="c", subcore_axis_name="s", num_cores=_NC,
        num_subcores=_NS)
    body = functools.partial(_sc_select_body, k, H // _LANES)
    nbmax = 1 << max(_LVL_BITS)
    return pl.kernel(
        body,
        out_type=jax.ShapeDtypeStruct((B, _LANES), jnp.int32),
        mesh=mesh,
        compiler_params=pltpu.CompilerParams(needs_layout_passes=False),
        scratch_types=[
            pltpu.VMEM((H,), jnp.float32),
            pltpu.VMEM((H,), jnp.float32),
            pltpu.VMEM((nbmax * _LANES,), jnp.int32),
            pltpu.VMEM((nbmax * _LANES,), jnp.int32),
            pltpu.VMEM((_LANES,), jnp.int32),
        ] + [pltpu.SemaphoreType.DMA] * _NCHUNK,
    )(x)


def _apply_body(thr_ref, x_ref, sparse_ref, mask_ref):
    x = x_ref[...]
    u = lax.bitcast_convert_type(jnp.abs(x), jnp.int32)
    keep = u >= thr_ref[:, 0:1]
    mask_ref[...] = keep.astype(jnp.int8)
    sparse_ref[...] = jnp.where(keep, x, 0.0)


def _tc_apply(x, thr, rows):
    B, H = x.shape
    return pl.pallas_call(
        _apply_body,
        grid=(B // rows,),
        in_specs=[
            pl.BlockSpec((rows, _LANES), lambda i: (i, 0)),
            pl.BlockSpec((rows, H), lambda i: (i, 0)),
        ],
        out_specs=[
            pl.BlockSpec((rows, H), lambda i: (i, 0)),
            pl.BlockSpec((rows, H), lambda i: (i, 0)),
        ],
        out_shape=[
            jax.ShapeDtypeStruct((B, H), jnp.float32),
            jax.ShapeDtypeStruct((B, H), jnp.int8),
        ],
    )(thr, x)


def _select_body(x_ref, sparse_ref, mask_ref, *, k):
    x = x_ref[...]
    u = lax.bitcast_convert_type(jnp.abs(x), jnp.int32)

    def step(i, p):
        cand = p | (jnp.int32(1) << (30 - i))
        cnt = jnp.sum((u >= cand).astype(jnp.int32), axis=1, keepdims=True)
        return jnp.where(cnt >= k, cand, p)

    p0 = jnp.zeros((x.shape[0], 1), jnp.int32)
    thr = lax.fori_loop(0, 31, step, p0)
    keep = u >= thr
    mask_ref[...] = keep
    sparse_ref[...] = jnp.where(keep, x, 0.0)


def _tc_only(flat, k):
    B, H = flat.shape
    rows = 8 if B % 8 == 0 else 1
    return pl.pallas_call(
        functools.partial(_select_body, k=k),
        grid=(B // rows,),
        in_specs=[pl.BlockSpec((rows, H), lambda i: (i, 0))],
        out_specs=[
            pl.BlockSpec((rows, H), lambda i: (i, 0)),
            pl.BlockSpec((rows, H), lambda i: (i, 0)),
        ],
        out_shape=[
            jax.ShapeDtypeStruct((B, H), jnp.float32),
            jax.ShapeDtypeStruct((B, H), jnp.bool_),
        ],
    )(flat)


def kernel(x):
    flat = x if x.ndim == 2 else x.reshape(x.shape[0], -1)
    B, H = flat.shape
    k = max(1, int(H * 10.0 / 100.0))
    if B == 2 * _NW and H % (_LANES * _NCHUNK) == 0 and B % 32 == 0:
        thr = _sc_select(flat, k)
        sparse, mask8 = _tc_apply(flat, thr, 32)
        mask = mask8.astype(jnp.bool_)
    else:
        sparse, mask = _tc_only(flat, k)
    return sparse.reshape(x.shape), mask.reshape(x.shape)


# speedup vs baseline: 1.2339x; 1.0290x over previous
import functools

import jax
import jax.numpy as jnp
from jax import lax
from jax.experimental import pallas as pl
from jax.experimental.pallas import tpu as pltpu
from jax.experimental.pallas import tpu_sc as plsc

_NC, _NS, _LANES = 2, 16, 16
_NW = _NC * _NS

_LVL_BITS = (8, 8, 8, 7)
_LVL_SHIFTS = (23, 15, 7, 0)
_NCHUNK = 4


def _bucket16(v, shift, nbits):
    m16 = ((1 << nbits) - 1) << 4
    if shift >= 4:
        return (v >> (shift - 4)) & jnp.int32(m16)
    return (v << (4 - shift)) & jnp.int32(m16)


def _sc_select_body(k, nvec, x_hbm, thr_hbm, xbuf0, xbuf1, hist0, hist1,
                    tvec, *sems):
    lane = lax.iota(jnp.int32, _LANES)
    ones = jnp.full((_LANES,), 1, jnp.int32)
    zeros16 = jnp.zeros((_LANES,), jnp.int32)
    wid = lax.axis_index("c") * _NS + lax.axis_index("s")

    nelem = nvec * _LANES
    csize = nelem // _NCHUNK
    copies = []
    for c in range(_NCHUNK):
        copies.append(
            (pltpu.async_copy(x_hbm.at[wid * 2, pl.ds(c * csize, csize)],
                              xbuf0.at[pl.ds(c * csize, csize)], sems[c]),
             pltpu.async_copy(x_hbm.at[wid * 2 + 1, pl.ds(c * csize, csize)],
                              xbuf1.at[pl.ds(c * csize, csize)], sems[c])))

    kk0 = jnp.int32(k)
    kk1 = jnp.int32(k)
    p0 = jnp.int32(0)
    p1 = jnp.int32(0)
    for lvl in range(4):
        shift = _LVL_SHIFTS[lvl]
        nbits = _LVL_BITS[lvl]
        nb = 1 << nbits

        @plsc.parallel_loop(0, nb, unroll=8)
        def _(b):
            hist0[pl.ds(b * _LANES, _LANES)] = zeros16
            hist1[pl.ds(b * _LANES, _LANES)] = zeros16

        pfx0, pfx1 = p0, p1
        mlow = jnp.int32((1 << (31 - shift - nbits)) - 1)

        def span(i0v, i1v, lvl=lvl, shift=shift, nbits=nbits, nb=nb,
                 pfx0=pfx0, pfx1=pfx1, mlow=mlow):
            @plsc.parallel_loop(i0v, i1v, unroll=8)
            def _(i):
                v0 = plsc.bitcast(xbuf0[pl.ds(i * _LANES, _LANES)], jnp.int32)
                v1 = plsc.bitcast(xbuf1[pl.ds(i * _LANES, _LANES)], jnp.int32)
                i0 = _bucket16(v0, shift, nbits) + lane
                i1 = _bucket16(v1, shift, nbits) + lane
                if lvl == 0:
                    plsc.addupdate_scatter(hist0, [i0], ones)
                    plsc.addupdate_scatter(hist1, [i1], ones)
                else:
                    s0 = ((v0 >> (shift + nbits)) & mlow) == pfx0
                    s1 = ((v1 >> (shift + nbits)) & mlow) == pfx1
                    plsc.addupdate_scatter(hist0, [i0], ones, mask=s0)
                    plsc.addupdate_scatter(hist1, [i1], ones, mask=s1)

        if lvl == 0:
            cvec = csize // _LANES
            for c in range(_NCHUNK):
                copies[c][0].wait()
                copies[c][1].wait()
                span(c * cvec, (c + 1) * cvec)
        else:
            span(0, nvec)

        @plsc.parallel_loop(0, nb, unroll=4, carry=(zeros16, zeros16))
        def _(t, acc, nb=nb):
            a0, a1 = acc
            bb = nb - 1 - t
            a0 = a0 + hist0[pl.ds(bb * _LANES, _LANES)]
            a1 = a1 + hist1[pl.ds(bb * _LANES, _LANES)]
            hist0[pl.ds(bb * _LANES, _LANES)] = a0
            hist1[pl.ds(bb * _LANES, _LANES)] = a1
            return (a0, a1)

        def probe(hist, b):
            return jnp.sum(hist[pl.ds(b * _LANES, _LANES)])

        lo0 = jnp.int32(0)
        lo1 = jnp.int32(0)
        step = nb >> 1
        while step >= 1:
            c0 = lo0 + jnp.int32(step)
            c1 = lo1 + jnp.int32(step)
            sv0 = probe(hist0, jnp.minimum(c0, nb - 1))
            sv1 = probe(hist1, jnp.minimum(c1, nb - 1))
            lo0 = jnp.where((c0 <= nb - 1) & (sv0 >= kk0), c0, lo0)
            lo1 = jnp.where((c1 <= nb - 1) & (sv1 >= kk1), c1, lo1)
            step >>= 1
        a0 = probe(hist0, jnp.minimum(lo0 + 1, nb - 1))
        a1 = probe(hist1, jnp.minimum(lo1 + 1, nb - 1))
        kk0 = kk0 - jnp.where(lo0 + 1 <= nb - 1, a0, jnp.int32(0))
        kk1 = kk1 - jnp.where(lo1 + 1 <= nb - 1, a1, jnp.int32(0))
        p0 = (p0 << nbits) | lo0
        p1 = (p1 << nbits) | lo1
    tvec[...] = jnp.broadcast_to(p0, (_LANES,))
    pltpu.sync_copy(tvec, thr_hbm.at[wid * 2])
    tvec[...] = jnp.broadcast_to(p1, (_LANES,))
    pltpu.sync_copy(tvec, thr_hbm.at[wid * 2 + 1])


def _sc_select(x, k):
    B, H = x.shape
    mesh = plsc.VectorSubcoreMesh(
        core_axis_name="c", subcore_axis_name="s", num_cores=_NC,
        num_subcores=_NS)
    body = functools.partial(_sc_select_body, k, H // _LANES)
    nbmax = 1 << max(_LVL_BITS)
    return pl.kernel(
        body,
        out_type=jax.ShapeDtypeStruct((B, _LANES), jnp.int32),
        mesh=mesh,
        compiler_params=pltpu.CompilerParams(needs_layout_passes=False),
        scratch_types=[
            pltpu.VMEM((H,), jnp.float32),
            pltpu.VMEM((H,), jnp.float32),
            pltpu.VMEM((nbmax * _LANES,), jnp.int32),
            pltpu.VMEM((nbmax * _LANES,), jnp.int32),
            pltpu.VMEM((_LANES,), jnp.int32),
        ] + [pltpu.SemaphoreType.DMA] * _NCHUNK,
    )(x)


def _apply_body(thr_ref, x_ref, sparse_ref, mask_ref):
    x = x_ref[...]
    u = lax.bitcast_convert_type(jnp.abs(x), jnp.int32)
    keep = u >= thr_ref[:, 0:1]
    mask_ref[...] = keep.astype(jnp.int8)
    sparse_ref[...] = jnp.where(keep, x, 0.0)


def _tc_apply(x, thr, rows):
    B, H = x.shape
    return pl.pallas_call(
        _apply_body,
        grid=(B // rows,),
        in_specs=[
            pl.BlockSpec((rows, _LANES), lambda i: (i, 0)),
            pl.BlockSpec((rows, H), lambda i: (i, 0)),
        ],
        out_specs=[
            pl.BlockSpec((rows, H), lambda i: (i, 0)),
            pl.BlockSpec((rows, H), lambda i: (i, 0)),
        ],
        out_shape=[
            jax.ShapeDtypeStruct((B, H), jnp.float32),
            jax.ShapeDtypeStruct((B, H), jnp.int8),
        ],
    )(thr, x)


def _select_body(x_ref, sparse_ref, mask_ref, *, k):
    x = x_ref[...]
    u = lax.bitcast_convert_type(jnp.abs(x), jnp.int32)

    def step(i, p):
        cand = p | (jnp.int32(1) << (30 - i))
        cnt = jnp.sum((u >= cand).astype(jnp.int32), axis=1, keepdims=True)
        return jnp.where(cnt >= k, cand, p)

    p0 = jnp.zeros((x.shape[0], 1), jnp.int32)
    thr = lax.fori_loop(0, 31, step, p0)
    keep = u >= thr
    mask_ref[...] = keep
    sparse_ref[...] = jnp.where(keep, x, 0.0)


def _tc_only(flat, k):
    B, H = flat.shape
    rows = 8 if B % 8 == 0 else 1
    return pl.pallas_call(
        functools.partial(_select_body, k=k),
        grid=(B // rows,),
        in_specs=[pl.BlockSpec((rows, H), lambda i: (i, 0))],
        out_specs=[
            pl.BlockSpec((rows, H), lambda i: (i, 0)),
            pl.BlockSpec((rows, H), lambda i: (i, 0)),
        ],
        out_shape=[
            jax.ShapeDtypeStruct((B, H), jnp.float32),
            jax.ShapeDtypeStruct((B, H), jnp.bool_),
        ],
    )(flat)


def kernel(x):
    flat = x if x.ndim == 2 else x.reshape(x.shape[0], -1)
    B, H = flat.shape
    k = max(1, int(H * 10.0 / 100.0))
    if B == 2 * _NW and H % (_LANES * _NCHUNK) == 0 and B % 32 == 0:
        thr = _sc_select(flat, k)
        sparse, mask8 = _tc_apply(flat, thr, 32)
        mask = mask8.astype(jnp.bool_)
    else:
        sparse, mask = _tc_only(flat, k)
    return sparse.reshape(x.shape), mask.reshape(x.shape)
